# baseline (speedup 1.0000x reference)
import jax
import jax.numpy as jnp
from jax.experimental import pallas as pl
from jax.experimental.pallas import tpu as pltpu

_RB = 1024


def _lap_kernel(x_ref, halo_ref, out_ref):
    x = x_ref[0]
    h = halo_ref[0, 0:1, :]
    rb, w = x.shape
    is_first = pl.program_id(1) == 0
    is_last = pl.program_id(1) == pl.num_programs(1) - 1

    last_row = x[rb - 1 : rb, :]
    lane = jax.lax.broadcasted_iota(jnp.int32, (1, w), 1)
    cm = (lane < w - 1).astype(x.dtype)

    xc = jnp.concatenate([x[:, 1:], x[:, w - 1 :]], axis=1)
    hy = jnp.where(is_last, last_row, h)
    xd = jnp.concatenate([x[1:, :], hy], axis=0)
    xd1 = jnp.concatenate([xd[:, 1:], xd[:, w - 1 :]], axis=1)

    dx = x - xc
    dy = x - xd
    dxd = xd - xd1
    wv = dy - cm * (xc - xd1)

    pa = jnp.sum(dx * dx + dy * dy, axis=0, keepdims=True)
    pb = jnp.sum(dxd * wv, axis=0, keepdims=True)
    part = pa * 3.0 + pb * 2.0

    row0 = dx[0:1, :]
    part = part + jnp.where(is_first, -(row0 * row0), 0.0)
    rowl = dx[rb - 1 : rb, :]
    part = part + jnp.where(is_last, rowl * rowl, 0.0)
    out_ref[0] = part


def kernel(f):
    B, C, H, W = f.shape
    n = B * C
    nb = H // _RB
    x3 = f.reshape(n, H, W)

    out = pl.pallas_call(
        _lap_kernel,
        grid=(n, nb),
        in_specs=[
            pl.BlockSpec((1, _RB, W), lambda i, j: (i, j, 0)),
            pl.BlockSpec(
                (1, 8, W),
                lambda i, j: (i, jnp.minimum((j + 1) * (_RB // 8), H // 8 - 1), 0),
            ),
        ],
        out_specs=pl.BlockSpec((1, 1, W), lambda i, j: (i * nb + j, 0, 0)),
        out_shape=jax.ShapeDtypeStruct((n * nb, 1, W), f.dtype),
        compiler_params=pltpu.CompilerParams(
            dimension_semantics=("parallel", "arbitrary"),
        ),
    )(x3, x3)

    return 2.0 * jnp.sum(out)


# speedup vs baseline: 1.0386x; 1.0386x over previous
import jax
import jax.numpy as jnp
from jax.experimental import pallas as pl
from jax.experimental.pallas import tpu as pltpu

_RB = 1024


def _lap_kernel(x_ref, halo_ref, out_ref):
    x = x_ref[0]
    h = halo_ref[0, 0:1, :]
    rb, w = x.shape
    is_first = pl.program_id(1) == 0
    is_last = pl.program_id(1) == pl.num_programs(1) - 1

    last_row = x[rb - 1 : rb, :]

    xc = jnp.concatenate([x[:, 1:], x[:, w - 1 :]], axis=1)
    hy = jnp.where(is_last, last_row, h)
    xd = jnp.concatenate([x[1:, :], hy], axis=0)
    xd1 = jnp.concatenate([xd[:, 1:], xd[:, w - 1 :]], axis=1)

    dx = x - xc
    dy = x - xd
    dxd = xd - xd1
    s = (dx * dx + dy * dy) * 3.0 + dxd * (dx - dxd) * 2.0
    part = jnp.sum(s, axis=0, keepdims=True)

    row0 = dx[0:1, :]
    part = part + jnp.where(is_first, -(row0 * row0), 0.0)
    rowl = dx[rb - 1 : rb, :]
    part = part + jnp.where(is_last, rowl * rowl, 0.0)
    out_ref[0] = part


def kernel(f):
    B, C, H, W = f.shape
    n = B * C
    nb = H // _RB
    x3 = f.reshape(n, H, W)

    out = pl.pallas_call(
        _lap_kernel,
        grid=(n, nb),
        in_specs=[
            pl.BlockSpec((1, _RB, W), lambda i, j: (i, j, 0)),
            pl.BlockSpec(
                (1, 8, W),
                lambda i, j: (i, jnp.minimum((j + 1) * (_RB // 8), H // 8 - 1), 0),
            ),
        ],
        out_specs=pl.BlockSpec((1, 1, W), lambda i, j: (i * nb + j, 0, 0)),
        out_shape=jax.ShapeDtypeStruct((n * nb, 1, W), f.dtype),
        compiler_params=pltpu.CompilerParams(
            dimension_semantics=("parallel", "arbitrary"),
        ),
    )(x3, x3)

    return 2.0 * jnp.sum(out)


# speedup vs baseline: 1.0524x; 1.0133x over previous
import jax
import jax.numpy as jnp
from jax.experimental import pallas as pl
from jax.experimental.pallas import tpu as pltpu


def _lap_kernel(x_ref, out_ref):
    x = x_ref[0]
    rb, w = x.shape
    last_row = x[rb - 1 : rb, :]

    xc = jnp.concatenate([x[:, 1:], x[:, w - 1 :]], axis=1)
    xd = jnp.concatenate([x[1:, :], last_row], axis=0)
    xd1 = jnp.concatenate([xd[:, 1:], xd[:, w - 1 :]], axis=1)

    dx = x - xc
    dy = x - xd
    dxd = xd - xd1
    s = (dx * dx + dy * dy) * 3.0 + dxd * (dx - dxd) * 2.0
    part = jnp.sum(s, axis=0, keepdims=True)

    row0 = dx[0:1, :]
    rowl = dx[rb - 1 : rb, :]
    out_ref[0] = part - row0 * row0 + rowl * rowl


def kernel(f):
    B, C, H, W = f.shape
    n = B * C
    x3 = f.reshape(n, H, W)

    out = pl.pallas_call(
        _lap_kernel,
        grid=(n,),
        in_specs=[pl.BlockSpec((1, H, W), lambda i: (i, 0, 0))],
        out_specs=pl.BlockSpec((1, 1, W), lambda i: (i, 0, 0)),
        out_shape=jax.ShapeDtypeStruct((n, 1, W), f.dtype),
        compiler_params=pltpu.CompilerParams(
            dimension_semantics=("parallel",),
        ),
    )(x3)

    return 2.0 * jnp.sum(out)
